# baseline (speedup 1.0000x reference)
import functools

import jax
import jax.numpy as jnp
from jax import lax
from jax.experimental import pallas as pl
from jax.experimental.pallas import tpu as pltpu
from jax.experimental.pallas import tpu_sc as plsc

EMBED_DIM = 768
HIDDEN_DIM = 3072
NUM_EXPERTS = 8
TOP_K = 2

BT = 256
T_TOKENS = 2048
NPAIR = T_TOKENS * TOP_K
NG = NPAIR // BT + NUM_EXPERTS
NP_ROWS = NG * BT

NW = 32
GC = 96
TPW = T_TOKENS // NW


def _router_kernel(x_ref, wr_ref, br_ref, eidx_ref, pw_ref):
    logits = jnp.dot(x_ref[...], wr_ref[...],
                     preferred_element_type=jnp.float32) + br_ref[...]
    lane = lax.broadcasted_iota(jnp.int32, logits.shape, 1)
    m1 = jnp.max(logits, axis=1, keepdims=True)
    i1 = jnp.min(jnp.where(logits == m1, lane, NUM_EXPERTS), axis=1,
                 keepdims=True)
    l2 = jnp.where(lane == i1, -jnp.inf, logits)
    m2 = jnp.max(l2, axis=1, keepdims=True)
    i2 = jnp.min(jnp.where(l2 == m2, lane, NUM_EXPERTS), axis=1, keepdims=True)
    p2 = 1.0 / (1.0 + jnp.exp(m1 - m2))
    p1 = 1.0 - p2
    eidx_ref[...] = jnp.concatenate([i1, i2], axis=1)
    pw_ref[...] = jnp.concatenate([p1, p2], axis=1)


def _ffn_kernel(blk_e_ref, xg_ref, w_ref, w1_ref, b1_ref, w2_ref, b2_ref,
                y_ref):
    del blk_e_ref
    h = jnp.dot(xg_ref[...], w1_ref[0], preferred_element_type=jnp.float32)
    h = jnp.maximum(h + b1_ref[0], 0.0)
    y = jnp.dot(h, w2_ref[0], preferred_element_type=jnp.float32) + b2_ref[0]
    y_ref[...] = w_ref[0] * y


def _dispatch_kernel(x_hbm, tok_hbm, xg_hbm, idx_v, rows_v, sem):
    wid = lax.axis_index("s") * 2 + lax.axis_index("c")
    base = wid * (NP_ROWS // NW)
    pltpu.sync_copy(tok_hbm.at[wid], idx_v)
    for ci in range(NP_ROWS // NW // GC):
        pltpu.async_copy(x_hbm.at[idx_v.at[ci]], rows_v, sem).wait()
        pltpu.sync_copy(rows_v, xg_hbm.at[pl.ds(base + ci * GC, GC)])


def _combine_kernel(y_hbm, pos_hbm, out_hbm, p0_v, p1_v, buf0, buf1, sem):
    wid = lax.axis_index("s") * 2 + lax.axis_index("c")
    base = wid * TPW
    pltpu.sync_copy(pos_hbm.at[wid, 0], p0_v)
    pltpu.sync_copy(pos_hbm.at[wid, 1], p1_v)
    c0 = pltpu.async_copy(y_hbm.at[p0_v], buf0, sem)
    c1 = pltpu.async_copy(y_hbm.at[p1_v], buf1, sem)
    c0.wait()
    c1.wait()

    def row(r, carry):
        for j in range(EMBED_DIM // 16):
            sl = pl.ds(j * 16, 16)
            buf0[r, sl] = buf0[r, sl] + buf1[r, sl]
        return carry

    lax.fori_loop(0, TPW, row, 0)
    pltpu.sync_copy(buf0, out_hbm.at[pl.ds(base, TPW)])


def kernel(x, Wr, br, W1, b1, W2, b2):
    batch, seq, d = x.shape
    x_flat = x.reshape(-1, d)
    T = x_flat.shape[0]

    eidx, pw = pl.pallas_call(
        _router_kernel,
        grid=(1,),
        in_specs=[
            pl.BlockSpec((T, d), lambda i: (0, 0)),
            pl.BlockSpec((d, NUM_EXPERTS), lambda i: (0, 0)),
            pl.BlockSpec((1, NUM_EXPERTS), lambda i: (0, 0)),
        ],
        out_specs=[
            pl.BlockSpec((T, TOP_K), lambda i: (0, 0)),
            pl.BlockSpec((T, TOP_K), lambda i: (0, 0)),
        ],
        out_shape=[
            jax.ShapeDtypeStruct((T, TOP_K), jnp.int32),
            jax.ShapeDtypeStruct((T, TOP_K), jnp.float32),
        ],
    )(x_flat, Wr, br.reshape(1, NUM_EXPERTS))

    tok_pad, w_pad, blk_e, pos = _plan(eidx, pw)

    mesh = plsc.VectorSubcoreMesh(core_axis_name="c", subcore_axis_name="s")

    tok3 = tok_pad.reshape(NW, (NP_ROWS // NW) // GC, GC)
    xg = pl.kernel(
        _dispatch_kernel,
        mesh=mesh,
        out_type=jax.ShapeDtypeStruct((NP_ROWS, d), jnp.float32),
        scratch_types=[
            pltpu.VMEM(((NP_ROWS // NW) // GC, GC), jnp.int32),
            pltpu.VMEM((GC, d), jnp.float32),
            pltpu.SemaphoreType.DMA,
        ],
    )(x_flat, tok3)

    y = pl.pallas_call(
        _ffn_kernel,
        grid_spec=pltpu.PrefetchScalarGridSpec(
            num_scalar_prefetch=1,
            grid=(NG,),
            in_specs=[
                pl.BlockSpec((BT, d), lambda g, be: (g, 0)),
                pl.BlockSpec((1, BT, 1), lambda g, be: (g, 0, 0)),
                pl.BlockSpec((1, d, HIDDEN_DIM), lambda g, be: (be[g], 0, 0)),
                pl.BlockSpec((1, 1, HIDDEN_DIM), lambda g, be: (be[g], 0, 0)),
                pl.BlockSpec((1, HIDDEN_DIM, d), lambda g, be: (be[g], 0, 0)),
                pl.BlockSpec((1, 1, d), lambda g, be: (be[g], 0, 0)),
            ],
            out_specs=pl.BlockSpec((BT, d), lambda g, be: (g, 0)),
        ),
        out_shape=jax.ShapeDtypeStruct((NP_ROWS, d), jnp.float32),
    )(blk_e, xg, w_pad.reshape(NG, BT, 1), W1,
      b1.reshape(NUM_EXPERTS, 1, HIDDEN_DIM), W2,
      b2.reshape(NUM_EXPERTS, 1, EMBED_DIM))

    out = pl.kernel(
        _combine_kernel,
        mesh=mesh,
        out_type=jax.ShapeDtypeStruct((T, d), jnp.float32),
        scratch_types=[
            pltpu.VMEM((TPW,), jnp.int32),
            pltpu.VMEM((TPW,), jnp.int32),
            pltpu.VMEM((TPW, d), jnp.float32),
            pltpu.VMEM((TPW, d), jnp.float32),
            pltpu.SemaphoreType.DMA,
        ],
    )(y, pos)

    return out.reshape(batch, seq, d)


def _plan(eidx, pw):
    T = T_TOKENS
    e_flat = eidx.reshape(-1)
    w_flat = pw.reshape(-1)
    oh = (e_flat[:, None] == jnp.arange(NUM_EXPERTS)[None, :]).astype(jnp.int32)
    cnt_incl = jnp.cumsum(oh, axis=0)
    cnt_before = jnp.take_along_axis(cnt_incl, e_flat[:, None], 1)[:, 0] - 1
    counts = cnt_incl[-1]
    nb = (counts + BT - 1) // BT
    bounds = jnp.concatenate([jnp.zeros((1,), jnp.int32),
                              jnp.cumsum(nb)[:-1]]).astype(jnp.int32)
    pad_base = BT * bounds
    padpos = pad_base[e_flat] + cnt_before
    tok_pad = jnp.zeros((NP_ROWS,), jnp.int32).at[padpos].set(
        jnp.arange(NPAIR, dtype=jnp.int32) // TOP_K, unique_indices=True)
    w_pad = jnp.zeros((NP_ROWS,), jnp.float32).at[padpos].set(
        w_flat, unique_indices=True)
    blk_e = jnp.clip(
        jnp.searchsorted(bounds, jnp.arange(NG, dtype=jnp.int32),
                         side='right').astype(jnp.int32) - 1,
        0, NUM_EXPERTS - 1)
    pos = padpos.reshape(NW, TPW, TOP_K).transpose(0, 2, 1)
    return tok_pad, w_pad, blk_e, pos


# speedup vs baseline: 1.0037x; 1.0037x over previous
import functools

import jax
import jax.numpy as jnp
from jax import lax
from jax.experimental import pallas as pl
from jax.experimental.pallas import tpu as pltpu
from jax.experimental.pallas import tpu_sc as plsc

EMBED_DIM = 768
HIDDEN_DIM = 3072
NUM_EXPERTS = 8
TOP_K = 2

BT = 256
T_TOKENS = 2048
NPAIR = T_TOKENS * TOP_K
NG = NPAIR // BT + NUM_EXPERTS
NP_ROWS = NG * BT

NW = 32
GC = 48
RPW = NP_ROWS // NW
NCH = RPW // GC
NBUF = 3
TPW = T_TOKENS // NW


def _router_kernel(x_ref, wr_ref, br_ref, eidx_ref, pw_ref):
    logits = jnp.dot(x_ref[...], wr_ref[...],
                     preferred_element_type=jnp.float32) + br_ref[...]
    lane = lax.broadcasted_iota(jnp.int32, logits.shape, 1)
    m1 = jnp.max(logits, axis=1, keepdims=True)
    i1 = jnp.min(jnp.where(logits == m1, lane, NUM_EXPERTS), axis=1,
                 keepdims=True)
    l2 = jnp.where(lane == i1, -jnp.inf, logits)
    m2 = jnp.max(l2, axis=1, keepdims=True)
    i2 = jnp.min(jnp.where(l2 == m2, lane, NUM_EXPERTS), axis=1, keepdims=True)
    p2 = 1.0 / (1.0 + jnp.exp(m1 - m2))
    p1 = 1.0 - p2
    eidx_ref[...] = jnp.concatenate([i1, i2], axis=1)
    pw_ref[...] = jnp.concatenate([p1, p2], axis=1)


def _ffn_kernel(blk_e_ref, xg_ref, w_ref, w1_ref, b1_ref, w2_ref, b2_ref,
                y_ref):
    del blk_e_ref
    h = jnp.dot(xg_ref[...], w1_ref[0], preferred_element_type=jnp.float32)
    h = jnp.maximum(h + b1_ref[0], 0.0)
    y = jnp.dot(h, w2_ref[0], preferred_element_type=jnp.float32) + b2_ref[0]
    y_ref[...] = w_ref[0] * y


def _dispatch_kernel(x_hbm, tok_hbm, xg_hbm, idx_v, r0, r1, r2, gsem, wsem):
    wid = lax.axis_index("s") * 2 + lax.axis_index("c")
    base = wid * RPW
    pltpu.sync_copy(tok_hbm.at[wid], idx_v)
    bufs = (r0, r1, r2)
    g, w = [None] * NCH, [None] * NCH
    for b in range(min(NBUF, NCH)):
        g[b] = pltpu.async_copy(x_hbm.at[idx_v.at[b]], bufs[b], gsem)
    for ci in range(NCH):
        g[ci].wait()
        w[ci] = pltpu.async_copy(bufs[ci % NBUF],
                                 xg_hbm.at[pl.ds(base + ci * GC, GC)], wsem)
        nxt = ci + NBUF
        if nxt < NCH:
            w[ci].wait()
            g[nxt] = pltpu.async_copy(x_hbm.at[idx_v.at[nxt]],
                                      bufs[nxt % NBUF], gsem)
    for ci in range(NCH):
        if w[ci] is not None and (ci + NBUF >= NCH):
            w[ci].wait()


def _combine_kernel(y_hbm, pos_hbm, out_hbm, p0_v, p1_v, buf0, buf1, sem):
    wid = lax.axis_index("s") * 2 + lax.axis_index("c")
    base = wid * TPW
    pltpu.sync_copy(pos_hbm.at[wid, 0], p0_v)
    pltpu.sync_copy(pos_hbm.at[wid, 1], p1_v)
    c0 = pltpu.async_copy(y_hbm.at[p0_v], buf0, sem)
    c1 = pltpu.async_copy(y_hbm.at[p1_v], buf1, sem)
    c0.wait()
    c1.wait()

    def row(r, carry):
        for j in range(EMBED_DIM // 16):
            sl = pl.ds(j * 16, 16)
            buf0[r, sl] = buf0[r, sl] + buf1[r, sl]
        return carry

    lax.fori_loop(0, TPW, row, 0)
    pltpu.sync_copy(buf0, out_hbm.at[pl.ds(base, TPW)])


def kernel(x, Wr, br, W1, b1, W2, b2):
    batch, seq, d = x.shape
    x_flat = x.reshape(-1, d)
    T = x_flat.shape[0]

    eidx, pw = pl.pallas_call(
        _router_kernel,
        grid=(1,),
        in_specs=[
            pl.BlockSpec((T, d), lambda i: (0, 0)),
            pl.BlockSpec((d, NUM_EXPERTS), lambda i: (0, 0)),
            pl.BlockSpec((1, NUM_EXPERTS), lambda i: (0, 0)),
        ],
        out_specs=[
            pl.BlockSpec((T, TOP_K), lambda i: (0, 0)),
            pl.BlockSpec((T, TOP_K), lambda i: (0, 0)),
        ],
        out_shape=[
            jax.ShapeDtypeStruct((T, TOP_K), jnp.int32),
            jax.ShapeDtypeStruct((T, TOP_K), jnp.float32),
        ],
    )(x_flat, Wr, br.reshape(1, NUM_EXPERTS))

    tok_pad, w_pad, blk_e, pos = _plan(eidx, pw)

    mesh = plsc.VectorSubcoreMesh(core_axis_name="c", subcore_axis_name="s")

    tok3 = tok_pad.reshape(NW, NCH, GC)
    xg = pl.kernel(
        _dispatch_kernel,
        mesh=mesh,
        out_type=jax.ShapeDtypeStruct((NP_ROWS, d), jnp.float32),
        scratch_types=[
            pltpu.VMEM((NCH, GC), jnp.int32),
            pltpu.VMEM((GC, d), jnp.float32),
            pltpu.VMEM((GC, d), jnp.float32),
            pltpu.VMEM((GC, d), jnp.float32),
            pltpu.SemaphoreType.DMA,
            pltpu.SemaphoreType.DMA,
        ],
    )(x_flat, tok3)

    y = pl.pallas_call(
        _ffn_kernel,
        grid_spec=pltpu.PrefetchScalarGridSpec(
            num_scalar_prefetch=1,
            grid=(NG,),
            in_specs=[
                pl.BlockSpec((BT, d), lambda g, be: (g, 0)),
                pl.BlockSpec((1, BT, 1), lambda g, be: (g, 0, 0)),
                pl.BlockSpec((1, d, HIDDEN_DIM), lambda g, be: (be[g], 0, 0)),
                pl.BlockSpec((1, 1, HIDDEN_DIM), lambda g, be: (be[g], 0, 0)),
                pl.BlockSpec((1, HIDDEN_DIM, d), lambda g, be: (be[g], 0, 0)),
                pl.BlockSpec((1, 1, d), lambda g, be: (be[g], 0, 0)),
            ],
            out_specs=pl.BlockSpec((BT, d), lambda g, be: (g, 0)),
        ),
        out_shape=jax.ShapeDtypeStruct((NP_ROWS, d), jnp.float32),
    )(blk_e, xg, w_pad.reshape(NG, BT, 1), W1,
      b1.reshape(NUM_EXPERTS, 1, HIDDEN_DIM), W2,
      b2.reshape(NUM_EXPERTS, 1, EMBED_DIM))

    out = pl.kernel(
        _combine_kernel,
        mesh=mesh,
        out_type=jax.ShapeDtypeStruct((T, d), jnp.float32),
        scratch_types=[
            pltpu.VMEM((TPW,), jnp.int32),
            pltpu.VMEM((TPW,), jnp.int32),
            pltpu.VMEM((TPW, d), jnp.float32),
            pltpu.VMEM((TPW, d), jnp.float32),
            pltpu.SemaphoreType.DMA,
        ],
    )(y, pos)

    return out.reshape(batch, seq, d)


def _plan(eidx, pw):
    T = T_TOKENS
    e_flat = eidx.reshape(-1)
    w_flat = pw.reshape(-1)
    oh = (e_flat[:, None] == jnp.arange(NUM_EXPERTS)[None, :]).astype(jnp.int32)
    cnt_incl = jnp.cumsum(oh, axis=0)
    cnt_before = jnp.take_along_axis(cnt_incl, e_flat[:, None], 1)[:, 0] - 1
    counts = cnt_incl[-1]
    nb = (counts + BT - 1) // BT
    bounds = jnp.concatenate([jnp.zeros((1,), jnp.int32),
                              jnp.cumsum(nb)[:-1]]).astype(jnp.int32)
    pad_base = BT * bounds
    padpos = pad_base[e_flat] + cnt_before
    tok_pad = jnp.zeros((NP_ROWS,), jnp.int32).at[padpos].set(
        jnp.arange(NPAIR, dtype=jnp.int32) // TOP_K, unique_indices=True)
    w_pad = jnp.zeros((NP_ROWS,), jnp.float32).at[padpos].set(
        w_flat, unique_indices=True)
    blk_e = jnp.clip(
        jnp.searchsorted(bounds, jnp.arange(NG, dtype=jnp.int32),
                         side='right').astype(jnp.int32) - 1,
        0, NUM_EXPERTS - 1)
    pos = padpos.reshape(NW, TPW, TOP_K).transpose(0, 2, 1)
    return tok_pad, w_pad, blk_e, pos
